# Initial kernel scaffold; baseline (speedup 1.0000x reference)
import jax
import jax.numpy as jnp
from jax.experimental import pallas as pl


def kernel(sample, entity_embedding, relation_embedding):
    raise NotImplementedError("write your pallas kernel here")


# speedup vs baseline: 1.0557x; 1.0557x over previous
import functools

import jax
import jax.numpy as jnp
from jax import lax
from jax.experimental import pallas as pl
from jax.experimental.pallas import tpu as pltpu
from jax.experimental.pallas import tpu_sc as plsc

GAMMA = 12.0
EMB_RANGE = 0.21875
PI = 3.141592653589793
PHASE_SCALE = PI / EMB_RANGE

B = 4096
D = 64
ED = 128
NC, NS = 2, 16
NW = NC * NS
BPW = B // NW


def _gather_body(ent_hbm, rel_hbm, hidx_hbm, ridx_hbm, tidx_hbm,
                 hout_hbm, rout_hbm, tout_hbm,
                 hidx_v, ridx_v, tidx_v, hrows_v, rrows_v, trows_v, sem):
    wid = lax.axis_index("s") * NC + lax.axis_index("c")
    base = wid * BPW
    pltpu.sync_copy(hidx_hbm.at[pl.ds(base, BPW)], hidx_v)
    pltpu.sync_copy(ridx_hbm.at[pl.ds(base, BPW)], ridx_v)
    pltpu.sync_copy(tidx_hbm.at[pl.ds(base, BPW)], tidx_v)
    c1 = pltpu.async_copy(ent_hbm.at[hidx_v], hrows_v, sem)
    c2 = pltpu.async_copy(rel_hbm.at[ridx_v], rrows_v, sem)
    c3 = pltpu.async_copy(ent_hbm.at[tidx_v], trows_v, sem)
    c1.wait()
    c2.wait()
    c3.wait()
    pltpu.sync_copy(hrows_v, hout_hbm.at[pl.ds(base, BPW)])
    pltpu.sync_copy(rrows_v, rout_hbm.at[pl.ds(base, BPW)])
    pltpu.sync_copy(trows_v, tout_hbm.at[pl.ds(base, BPW)])


def _score_body(h_ref, r_ref, t_ref, o_ref):
    re_h = h_ref[:, :D]
    im_h = h_ref[:, D:]
    re_t = t_ref[:, :D]
    im_t = t_ref[:, D:]
    ph = r_ref[:, :D] * PHASE_SCALE
    re_r = jnp.cos(ph)
    im_r = jnp.sin(ph)
    re_s = re_h * re_r - im_h * im_r - re_t
    im_s = re_h * im_r + im_h * re_r - im_t
    mag = jnp.sqrt(re_s * re_s + im_s * im_s)
    o_ref[...] = GAMMA - jnp.sum(mag, axis=1, keepdims=True)


def kernel(sample, entity_embedding, relation_embedding):
    sample = sample.astype(jnp.int32)
    hidx = sample[:, 0]
    ridx = sample[:, 1]
    tidx = sample[:, 2]
    f32 = jnp.float32
    relation_padded = jnp.pad(relation_embedding, ((0, 0), (0, ED - D)))
    mesh = plsc.VectorSubcoreMesh(core_axis_name="c", subcore_axis_name="s")

    gather = pl.kernel(
        _gather_body,
        out_type=(jax.ShapeDtypeStruct((B, ED), f32),
                  jax.ShapeDtypeStruct((B, ED), f32),
                  jax.ShapeDtypeStruct((B, ED), f32)),
        mesh=mesh,
        scratch_types=[
            pltpu.VMEM((BPW,), jnp.int32),
            pltpu.VMEM((BPW,), jnp.int32),
            pltpu.VMEM((BPW,), jnp.int32),
            pltpu.VMEM((BPW, ED), f32),
            pltpu.VMEM((BPW, ED), f32),
            pltpu.VMEM((BPW, ED), f32),
            pltpu.SemaphoreType.DMA,
        ],
    )
    hrows, rrows, trows = gather(entity_embedding, relation_padded,
                                 hidx, ridx, tidx)

    nblk = 4
    score = pl.pallas_call(
        _score_body,
        out_shape=jax.ShapeDtypeStruct((B, 1), f32),
        grid=(nblk,),
        in_specs=[
            pl.BlockSpec((B // nblk, ED), lambda i: (i, 0)),
            pl.BlockSpec((B // nblk, ED), lambda i: (i, 0)),
            pl.BlockSpec((B // nblk, ED), lambda i: (i, 0)),
        ],
        out_specs=pl.BlockSpec((B // nblk, 1), lambda i: (i, 0)),
    )(hrows, rrows, trows)
    return score


# speedup vs baseline: 1.1136x; 1.0548x over previous
import jax
import jax.numpy as jnp
from jax import lax
from jax.experimental import pallas as pl
from jax.experimental.pallas import tpu as pltpu
from jax.experimental.pallas import tpu_sc as plsc

GAMMA = 12.0
EMB_RANGE = 0.21875
PI = 3.141592653589793
PHASE_SCALE = PI / EMB_RANGE

B = 4096
D = 64
ED = 128
NREL_PAD = 1024
NC, NS = 2, 16
NW = NC * NS
BPW = B // NW


def _phase_table_body(r_ref, o_ref):
    ph = r_ref[...] * PHASE_SCALE
    o_ref[:, :D] = jnp.cos(ph)
    o_ref[:, D:] = jnp.sin(ph)


def _gather_body(ent_hbm, cs_hbm, hidx_hbm, tidx_hbm, ridx_hbm,
                 hout_hbm, tout_hbm, cout_hbm,
                 hidx_v, tidx_v, ridx_v, hbuf, tbuf, rbuf,
                 s1, s2, s3, s4, s5, s6):
    wid = lax.axis_index("s") * NC + lax.axis_index("c")
    base = wid * BPW
    pltpu.sync_copy(hidx_hbm.at[pl.ds(base, BPW)], hidx_v)
    pltpu.sync_copy(tidx_hbm.at[pl.ds(base, BPW)], tidx_v)
    pltpu.sync_copy(ridx_hbm.at[pl.ds(base, BPW)], ridx_v)
    gh = pltpu.async_copy(ent_hbm.at[hidx_v], hbuf, s1)
    gt = pltpu.async_copy(ent_hbm.at[tidx_v], tbuf, s2)
    gr = pltpu.async_copy(cs_hbm.at[ridx_v], rbuf, s3)
    gh.wait()
    wh = pltpu.async_copy(hbuf, hout_hbm.at[pl.ds(base, BPW)], s4)
    gt.wait()
    wt = pltpu.async_copy(tbuf, tout_hbm.at[pl.ds(base, BPW)], s5)
    gr.wait()
    wr = pltpu.async_copy(rbuf, cout_hbm.at[pl.ds(base, BPW)], s6)
    wh.wait()
    wt.wait()
    wr.wait()


def _score_body(h_ref, t_ref, c_ref, o_ref):
    re_h = h_ref[:, :D]
    im_h = h_ref[:, D:]
    re_t = t_ref[:, :D]
    im_t = t_ref[:, D:]
    re_r = c_ref[:, :D]
    im_r = c_ref[:, D:]
    re_s = re_h * re_r - im_h * im_r - re_t
    im_s = re_h * im_r + im_h * re_r - im_t
    mag = jnp.sqrt(re_s * re_s + im_s * im_s)
    o_ref[...] = GAMMA - jnp.sum(mag, axis=1, keepdims=True)


def kernel(sample, entity_embedding, relation_embedding):
    sample = sample.astype(jnp.int32)
    hidx = sample[:, 0]
    tidx = sample[:, 2]
    ridx = sample[:, 1]
    f32 = jnp.float32
    nrel = relation_embedding.shape[0]
    rel_pad = jnp.pad(relation_embedding, ((0, NREL_PAD - nrel), (0, 0)))

    cossin = pl.pallas_call(
        _phase_table_body,
        out_shape=jax.ShapeDtypeStruct((NREL_PAD, ED), f32),
    )(rel_pad)

    mesh = plsc.VectorSubcoreMesh(core_axis_name="c", subcore_axis_name="s")
    gather = pl.kernel(
        _gather_body,
        out_type=(jax.ShapeDtypeStruct((B, ED), f32),
                  jax.ShapeDtypeStruct((B, ED), f32),
                  jax.ShapeDtypeStruct((B, ED), f32)),
        mesh=mesh,
        scratch_types=[
            pltpu.VMEM((BPW,), jnp.int32),
            pltpu.VMEM((BPW,), jnp.int32),
            pltpu.VMEM((BPW,), jnp.int32),
            pltpu.VMEM((BPW, ED), f32),
            pltpu.VMEM((BPW, ED), f32),
            pltpu.VMEM((BPW, ED), f32),
            pltpu.SemaphoreType.DMA,
            pltpu.SemaphoreType.DMA,
            pltpu.SemaphoreType.DMA,
            pltpu.SemaphoreType.DMA,
            pltpu.SemaphoreType.DMA,
            pltpu.SemaphoreType.DMA,
        ],
    )
    hrows, trows, csrows = gather(entity_embedding, cossin, hidx, tidx, ridx)

    nblk = 4
    blk = B // nblk
    score = pl.pallas_call(
        _score_body,
        out_shape=jax.ShapeDtypeStruct((B, 1), f32),
        grid=(nblk,),
        in_specs=[
            pl.BlockSpec((blk, ED), lambda i: (i, 0)),
            pl.BlockSpec((blk, ED), lambda i: (i, 0)),
            pl.BlockSpec((blk, ED), lambda i: (i, 0)),
        ],
        out_specs=pl.BlockSpec((blk, 1), lambda i: (i, 0)),
    )(hrows, trows, csrows)
    return score
